# baseline (speedup 1.0000x reference)
import functools

import jax
import jax.numpy as jnp
from jax import lax
from jax.experimental import pallas as pl
from jax.experimental.pallas import tpu as pltpu
from jax.experimental.pallas import tpu_sc as plsc

N = 10000
E = 160000
D = 256
ED = 16
B = 64

NPAD = 10240
CHUNK = 128
NCHUNKS = E // CHUNK
NW = 32
ZROWS = 2 * NPAD // NW


def _sc_body(x_hbm, e_hbm, src_hbm, dst_hbm, out_hbm,
             srcbuf, dstbuf, idxbuf, xbuf, ebuf):
    c = lax.axis_index("c")
    s = lax.axis_index("s")
    w = s * 2 + c

    zero16 = jnp.zeros((16,), jnp.float32)

    def zrow(r, carry):
        for k in range(16):
            ebuf[r, pl.ds(k * 16, 16)] = zero16
        return carry

    lax.fori_loop(0, CHUNK, zrow, 0)
    for i in range(ZROWS // CHUNK):
        pltpu.sync_copy(
            ebuf, out_hbm.at[pl.ds(c * NPAD + s * ZROWS + i * CHUNK, CHUNK)])
    plsc.subcore_barrier()

    my_lo = (w * NCHUNKS) // NW * 0
    my_hi = jnp.where(w == 0, NCHUNKS, 0)
    base = c * NPAD

    def chunk_body(ch, carry):
        ebase = ch * CHUNK
        pltpu.sync_copy(src_hbm.at[pl.ds(ebase, CHUNK)], srcbuf)
        pltpu.sync_copy(dst_hbm.at[pl.ds(ebase, CHUNK)], dstbuf)
        for k in range(CHUNK // 16):
            idxbuf[pl.ds(k * 16, 16)] = dstbuf[pl.ds(k * 16, 16)] + base
        pltpu.sync_copy(x_hbm.at[srcbuf], xbuf)
        pltpu.sync_copy(e_hbm.at[pl.ds(ebase, CHUNK)], ebuf)

        def crow(r, cc):
            for k in range(16):
                xs = xbuf[r, pl.ds(k * 16, 16)]
                es = ebuf[r, pl.ds(k * 16, 16)]
                xbuf[r, pl.ds(k * 16, 16)] = jnp.maximum(xs + es, 0.0)
            return cc

        lax.fori_loop(0, CHUNK, crow, 0)
        pltpu.sync_copy(xbuf, out_hbm.at[idxbuf], add=True)
        return carry

    lax.fori_loop(my_lo, my_hi, chunk_body, 0)


_sc_messages_call = pl.kernel(
    _sc_body,
    out_type=jax.ShapeDtypeStruct((2 * NPAD, D), jnp.float32),
    mesh=plsc.VectorSubcoreMesh(core_axis_name="c", subcore_axis_name="s"),
    scratch_types=[
        pltpu.VMEM((CHUNK,), jnp.int32),
        pltpu.VMEM((CHUNK,), jnp.int32),
        pltpu.VMEM((CHUNK,), jnp.int32),
        pltpu.VMEM((CHUNK, D), jnp.float32),
        pltpu.VMEM((CHUNK, D), jnp.float32),
    ],
)


EB = 2000


def _elin_body(ea_ref, we1_ref, be1_ref, we2_ref, be2_ref, e1_ref, e2_ref):
    a = ea_ref[...]
    e1_ref[...] = jnp.dot(a, we1_ref[...],
                          preferred_element_type=jnp.float32) + be1_ref[...]
    e2_ref[...] = jnp.dot(a, we2_ref[...],
                          preferred_element_type=jnp.float32) + be2_ref[...]


def _elin2(edge_attr, We1, be1, We2, be2):
    grid = E // EB
    return pl.pallas_call(
        _elin_body,
        grid=(grid,),
        in_specs=[
            pl.BlockSpec((EB, ED), lambda i: (i, 0)),
            pl.BlockSpec((ED, D), lambda i: (0, 0)),
            pl.BlockSpec((1, D), lambda i: (0, 0)),
            pl.BlockSpec((ED, D), lambda i: (0, 0)),
            pl.BlockSpec((1, D), lambda i: (0, 0)),
        ],
        out_specs=[
            pl.BlockSpec((EB, D), lambda i: (i, 0)),
            pl.BlockSpec((EB, D), lambda i: (i, 0)),
        ],
        out_shape=[
            jax.ShapeDtypeStruct((E, D), jnp.float32),
            jax.ShapeDtypeStruct((E, D), jnp.float32),
        ],
    )(edge_attr, We1, be1.reshape(1, D), We2, be2.reshape(1, D))


NB = 400


def _node_body(relu, eps_ref, x_ref, agg0_ref, agg1_ref, w_ref, b_ref, h_ref):
    scale = 1.0 + eps_ref[0]
    pre = scale * x_ref[...] + (agg0_ref[...] + agg1_ref[...])
    h = jnp.dot(pre, w_ref[...],
                preferred_element_type=jnp.float32) + b_ref[...]
    if relu:
        h = jnp.maximum(h, 0.0)
    h_ref[...] = h


def _node_update(x, agg_pad, W, b, eps, relu):
    grid = N // NB
    return pl.pallas_call(
        functools.partial(_node_body, relu),
        grid=(grid,),
        in_specs=[
            pl.BlockSpec(memory_space=pltpu.SMEM),
            pl.BlockSpec((NB, D), lambda i: (i, 0)),
            pl.BlockSpec((NB, D), lambda i: (i, 0)),
            pl.BlockSpec((NB, D), lambda i: (i + NPAD // NB, 0)),
            pl.BlockSpec((D, D), lambda i: (0, 0)),
            pl.BlockSpec((1, D), lambda i: (0, 0)),
        ],
        out_specs=pl.BlockSpec((NB, D), lambda i: (i, 0)),
        out_shape=jax.ShapeDtypeStruct((N, D), jnp.float32),
    )(eps.reshape(1), x, agg_pad, agg_pad, W, b.reshape(1, D))


PB = 400


def _pool_body(h_ref, bi_ref, wh1_ref, bh1_ref, wh2_ref, bh2_ref, out_ref,
               sums, cnts):
    i = pl.program_id(0)

    @pl.when(i == 0)
    def _():
        sums[...] = jnp.zeros_like(sums)
        cnts[...] = jnp.zeros_like(cnts)

    bidx = bi_ref[0, 0, :]
    seg = lax.broadcasted_iota(jnp.int32, (B, PB), 0)
    maskf = jnp.where(bidx[None, :] == seg, 1.0, 0.0).astype(jnp.float32)
    sums[...] += jnp.dot(maskf, h_ref[...],
                         preferred_element_type=jnp.float32)
    cnts[...] += jnp.sum(maskf, axis=1, keepdims=True)

    @pl.when(i == pl.num_programs(0) - 1)
    def _():
        pooled = sums[...] / jnp.maximum(cnts[...], 1.0)
        z = jnp.maximum(
            jnp.dot(pooled, wh1_ref[...],
                    preferred_element_type=jnp.float32) + bh1_ref[...], 0.0)
        logits = jnp.dot(z, wh2_ref[...],
                         preferred_element_type=jnp.float32) + bh2_ref[...]
        out_ref[...] = jax.nn.sigmoid(logits)


def _pool_head(h, batch_idx, Wh1, bh1, Wh2, bh2):
    grid = N // PB
    bi = batch_idx.reshape(grid, 1, PB)
    return pl.pallas_call(
        _pool_body,
        grid=(grid,),
        in_specs=[
            pl.BlockSpec((PB, D), lambda i: (i, 0)),
            pl.BlockSpec((1, 1, PB), lambda i: (i, 0, 0)),
            pl.BlockSpec((D, 10), lambda i: (0, 0)),
            pl.BlockSpec((1, 10), lambda i: (0, 0)),
            pl.BlockSpec((10, 1), lambda i: (0, 0)),
            pl.BlockSpec((1, 1), lambda i: (0, 0)),
        ],
        out_specs=pl.BlockSpec((B, 1), lambda i: (0, 0)),
        out_shape=jax.ShapeDtypeStruct((B, 1), jnp.float32),
        scratch_shapes=[
            pltpu.VMEM((B, D), jnp.float32),
            pltpu.VMEM((B, 1), jnp.float32),
        ],
    )(h, bi, Wh1, bh1.reshape(1, 10), Wh2, bh2.reshape(1, 1))


def kernel(x, edge_index, edge_attr, batch_idx, W1, b1, We1, be1, eps1,
           W2, b2, We2, be2, eps2, Wh1, bh1, Wh2, bh2):
    src = edge_index[0]
    dst = edge_index[1]
    e1, e2 = _elin2(edge_attr, We1, be1, We2, be2)

    def _jnp_agg(xx, ee):
        m = jax.nn.relu(jnp.take(xx, src, axis=0) + ee)
        a = jax.ops.segment_sum(m, dst, num_segments=NPAD)
        return jnp.concatenate([a, jnp.zeros((NPAD, D), jnp.float32)], 0)

    agg1 = _jnp_agg(x, e1)
    h1 = _node_update(x, agg1, W1, b1, eps1, relu=True)
    agg2 = _jnp_agg(h1, e2)
    h2 = _node_update(h1, agg2, W2, b2, eps2, relu=False)
    return _pool_head(h2, batch_idx, Wh1, bh1, Wh2, bh2)
